# baseline (speedup 1.0000x reference)
import jax
import jax.numpy as jnp
from jax.experimental import pallas as pl
from functools import partial


def _leaky(x):
    return jnp.where(x >= 0, x, 0.2 * x)


def _elu(x):
    return jnp.where(x > 0, x, jnp.exp(jnp.minimum(x, 0.0)) - 1.0)


def _pair_attn(a_self_src, a_self_dst, a_part_src):
    e_self = _leaky(a_self_src + a_self_dst)
    e_part = _leaky(a_part_src + a_self_dst)
    m = jnp.maximum(e_self, e_part)
    ex_s = jnp.exp(e_self - m)
    ex_p = jnp.exp(e_part - m)
    s = ex_s + ex_p + 1e-16
    return ex_s / s, ex_p / s


def _fused_kernel(x_ref, codes_ref, w1_ref, as1_ref, ad1_ref, b1_ref,
                  w2_ref, as2_ref, ad2_ref, b2_ref, e4_ref, out_ref):
    x = x_ref[...]
    codes = codes_ref[...]
    L = codes.shape[1]
    stem_m = (codes != 0).astype(jnp.float32)
    cnt = jnp.sum(stem_m, axis=1)
    stem_cnt = jnp.maximum(cnt, 1.0)
    loop_cnt = jnp.maximum(L - cnt, 1.0)
    stem_sum = jnp.sum(x * stem_m, axis=1)
    loop_sum = jnp.sum(x, axis=1) - stem_sum
    S = stem_sum / stem_cnt
    Lp = loop_sum / loop_cnt

    w1 = w1_ref[...]
    a_s1 = as1_ref[...]
    a_d1 = ad1_ref[...]
    e4 = e4_ref[...]
    mm = partial(jnp.dot, preferred_element_type=jnp.float32)

    hS = mm(S, w1)
    hL = mm(Lp, w1)
    asS = mm(hS, a_s1)
    adS = mm(hS, a_d1)
    asL = mm(hL, a_s1)
    adL = mm(hL, a_d1)
    aS_self, aS_part = _pair_attn(asS, adS, asL)
    aL_self, aL_part = _pair_attn(asL, adL, asS)
    b1 = b1_ref[...]
    outS = _elu(mm(aS_self, e4) * hS + mm(aS_part, e4) * hL + b1)
    outL = _elu(mm(aL_self, e4) * hL + mm(aL_part, e4) * hS + b1)

    w2 = w2_ref[...]
    a_s2 = as2_ref[...]
    a_d2 = ad2_ref[...]
    h2S = mm(outS, w2)
    h2L = mm(outL, w2)
    as2S = mm(h2S, a_s2)
    ad2S = mm(h2S, a_d2)
    as2L = mm(h2L, a_s2)
    ad2L = mm(h2L, a_d2)
    aS2_self, aS2_part = _pair_attn(as2S, ad2S, as2L)
    aL2_self, aL2_part = _pair_attn(as2L, ad2L, as2S)
    b2 = b2_ref[...]
    out_ref[:, 0, :] = aS2_self * h2S + aS2_part * h2L + b2
    out_ref[:, 1, :] = aL2_self * h2L + aL2_part * h2S + b2


def kernel(rna_node_features, rna_batch_idx, rna_dot_bracket_codes,
           W1, a_src1, a_dst1, b1, W2, a_src2, a_dst2, b2):
    B, L = rna_dot_bracket_codes.shape
    D = rna_node_features.shape[1]
    heads, out1 = a_src1.shape
    x3 = rna_node_features.reshape(B, L, D)

    eyeh = jnp.eye(heads, dtype=jnp.float32)
    A_s1 = (eyeh[:, None, :] * a_src1[:, :, None]).reshape(heads * out1, heads)
    A_d1 = (eyeh[:, None, :] * a_dst1[:, :, None]).reshape(heads * out1, heads)
    E4 = jnp.repeat(eyeh, out1, axis=1)

    Bblk = 200
    grid = (B // Bblk,)

    out = pl.pallas_call(
        _fused_kernel,
        grid=grid,
        in_specs=[
            pl.BlockSpec((Bblk, L, D), lambda i: (i, 0, 0)),
            pl.BlockSpec((Bblk, L, 1), lambda i: (i, 0, 0)),
            pl.BlockSpec((D, D), lambda i: (0, 0)),
            pl.BlockSpec((D, heads), lambda i: (0, 0)),
            pl.BlockSpec((D, heads), lambda i: (0, 0)),
            pl.BlockSpec((1, D), lambda i: (0, 0)),
            pl.BlockSpec((D, D), lambda i: (0, 0)),
            pl.BlockSpec((D, 1), lambda i: (0, 0)),
            pl.BlockSpec((D, 1), lambda i: (0, 0)),
            pl.BlockSpec((1, D), lambda i: (0, 0)),
            pl.BlockSpec((heads, D), lambda i: (0, 0)),
        ],
        out_specs=pl.BlockSpec((Bblk, 2, D), lambda i: (i, 0, 0)),
        out_shape=jax.ShapeDtypeStruct((B, 2, D), jnp.float32),
    )(x3, rna_dot_bracket_codes.reshape(B, L, 1), W1, A_s1, A_d1, b1.reshape(1, D),
      W2, a_src2.reshape(D, 1), a_dst2.reshape(D, 1), b2.reshape(1, D), E4)

    motif_batch_idx = jnp.repeat(jnp.arange(B), 2)
    return (out.reshape(2 * B, D), motif_batch_idx)


# speedup vs baseline: 1.4743x; 1.4743x over previous
import jax
import jax.numpy as jnp
from jax.experimental import pallas as pl
from functools import partial


def _leaky(x):
    return jnp.where(x >= 0, x, 0.2 * x)


def _elu(x):
    return jnp.where(x > 0, x, jnp.exp(jnp.minimum(x, 0.0)) - 1.0)


def _pair_attn(a_self_src, a_self_dst, a_part_src):
    e_self = _leaky(a_self_src + a_self_dst)
    e_part = _leaky(a_part_src + a_self_dst)
    m = jnp.maximum(e_self, e_part)
    ex_s = jnp.exp(e_self - m)
    ex_p = jnp.exp(e_part - m)
    s = ex_s + ex_p + 1e-16
    return ex_s / s, ex_p / s


def _fused_kernel(x_ref, codes_ref, w1_ref, as1_ref, ad1_ref, b1_ref,
                  w2_ref, as2_ref, ad2_ref, b2_ref, e4_ref, out_ref):
    x = x_ref[...]
    codes = codes_ref[...]
    L = codes.shape[1]
    stem_m = (codes != 0).astype(jnp.float32)
    cnt = jnp.sum(stem_m, axis=1, keepdims=True)
    stem_cnt = jnp.maximum(cnt, 1.0)
    loop_cnt = jnp.maximum(L - cnt, 1.0)
    stem_sum = jnp.sum(x * stem_m[:, :, None], axis=1)
    loop_sum = jnp.sum(x, axis=1) - stem_sum
    S = stem_sum / stem_cnt
    Lp = loop_sum / loop_cnt

    w1 = w1_ref[...]
    a_s1 = as1_ref[...]
    a_d1 = ad1_ref[...]
    e4 = e4_ref[...]
    mm = partial(jnp.dot, preferred_element_type=jnp.float32)

    hS = mm(S, w1)
    hL = mm(Lp, w1)
    asS = mm(hS, a_s1)
    adS = mm(hS, a_d1)
    asL = mm(hL, a_s1)
    adL = mm(hL, a_d1)
    aS_self, aS_part = _pair_attn(asS, adS, asL)
    aL_self, aL_part = _pair_attn(asL, adL, asS)
    b1 = b1_ref[...]
    outS = _elu(mm(aS_self, e4) * hS + mm(aS_part, e4) * hL + b1)
    outL = _elu(mm(aL_self, e4) * hL + mm(aL_part, e4) * hS + b1)

    w2 = w2_ref[...]
    a_s2 = as2_ref[...]
    a_d2 = ad2_ref[...]
    h2S = mm(outS, w2)
    h2L = mm(outL, w2)
    as2S = mm(h2S, a_s2)
    ad2S = mm(h2S, a_d2)
    as2L = mm(h2L, a_s2)
    ad2L = mm(h2L, a_d2)
    aS2_self, aS2_part = _pair_attn(as2S, ad2S, as2L)
    aL2_self, aL2_part = _pair_attn(as2L, ad2L, as2S)
    b2 = b2_ref[...]
    out_ref[:, 0, :] = aS2_self * h2S + aS2_part * h2L + b2
    out_ref[:, 1, :] = aL2_self * h2L + aL2_part * h2S + b2


def kernel(rna_node_features, rna_batch_idx, rna_dot_bracket_codes,
           W1, a_src1, a_dst1, b1, W2, a_src2, a_dst2, b2):
    B, L = rna_dot_bracket_codes.shape
    D = rna_node_features.shape[1]
    heads, out1 = a_src1.shape
    x3 = rna_node_features.reshape(B, L, D)

    eyeh = jnp.eye(heads, dtype=jnp.float32)
    A_s1 = (eyeh[:, None, :] * a_src1[:, :, None]).reshape(heads * out1, heads)
    A_d1 = (eyeh[:, None, :] * a_dst1[:, :, None]).reshape(heads * out1, heads)
    E4 = jnp.repeat(eyeh, out1, axis=1)

    Bblk = 200
    grid = (B // Bblk,)

    out = pl.pallas_call(
        _fused_kernel,
        grid=grid,
        in_specs=[
            pl.BlockSpec((Bblk, L, D), lambda i: (i, 0, 0)),
            pl.BlockSpec((Bblk, L), lambda i: (i, 0)),
            pl.BlockSpec((D, D), lambda i: (0, 0)),
            pl.BlockSpec((D, heads), lambda i: (0, 0)),
            pl.BlockSpec((D, heads), lambda i: (0, 0)),
            pl.BlockSpec((1, D), lambda i: (0, 0)),
            pl.BlockSpec((D, D), lambda i: (0, 0)),
            pl.BlockSpec((D, 1), lambda i: (0, 0)),
            pl.BlockSpec((D, 1), lambda i: (0, 0)),
            pl.BlockSpec((1, D), lambda i: (0, 0)),
            pl.BlockSpec((heads, D), lambda i: (0, 0)),
        ],
        out_specs=pl.BlockSpec((Bblk, 2, D), lambda i: (i, 0, 0)),
        out_shape=jax.ShapeDtypeStruct((B, 2, D), jnp.float32),
    )(x3, rna_dot_bracket_codes, W1, A_s1, A_d1, b1.reshape(1, D),
      W2, a_src2.reshape(D, 1), a_dst2.reshape(D, 1), b2.reshape(1, D), E4)

    motif_batch_idx = jnp.repeat(jnp.arange(B), 2)
    return (out.reshape(2 * B, D), motif_batch_idx)


# speedup vs baseline: 3.2745x; 2.2210x over previous
import jax
import jax.numpy as jnp
from jax import lax
from jax.experimental import pallas as pl
from functools import partial

_G = 8


def _leaky(x):
    return jnp.where(x >= 0, x, 0.2 * x)


def _elu(x):
    return jnp.where(x > 0, x, jnp.exp(jnp.minimum(x, 0.0)) - 1.0)


def _pair_swap(v):
    even = (lax.broadcasted_iota(jnp.int32, v.shape, 0) & 1) == 0
    return jnp.where(even, jnp.roll(v, -1, axis=0), jnp.roll(v, 1, axis=0))


def _pair_attn(a_self_src, a_self_dst, a_part_src):
    e_self = _leaky(a_self_src + a_self_dst)
    e_part = _leaky(a_part_src + a_self_dst)
    m = jnp.maximum(e_self, e_part)
    ex_s = jnp.exp(e_self - m)
    ex_p = jnp.exp(e_part - m)
    s = ex_s + ex_p + 1e-16
    return ex_s / s, ex_p / s


def _fused_kernel(L, x_ref, codes_ref, w1_ref, as1_ref, ad1_ref, b1_ref,
                  w2_ref, as2_ref, ad2_ref, b2_ref, e4_ref, out_ref):
    x = x_ref[...]
    Gb, GL, D = x.shape
    codes = codes_ref[0]
    m = (codes != 0).astype(jnp.float32)
    mb = jnp.broadcast_to(m[:, :, None], x.shape)
    xm = x * mb

    parts = []
    for s in range(_G):
        sl = slice(s * L, (s + 1) * L)
        stems = jnp.sum(xm[:, sl, :], axis=1)
        tots = jnp.sum(x[:, sl, :], axis=1)
        cnts = jnp.sum(mb[:, sl, :], axis=1)
        Ss = stems / jnp.maximum(cnts, 1.0)
        Lps = (tots - stems) / jnp.maximum(L - cnts, 1.0)
        parts.append(Ss[:, None, :])
        parts.append(Lps[:, None, :])
    init = jnp.concatenate(parts, axis=1).reshape(Gb * 2 * _G, D)

    w1 = w1_ref[...]
    a_s1 = as1_ref[...]
    a_d1 = ad1_ref[...]
    e4 = e4_ref[...]
    mm = partial(jnp.dot, preferred_element_type=jnp.float32)

    h = mm(init, w1)
    hp = _pair_swap(h)
    a_self_s = mm(h, a_s1)
    a_self_d = mm(h, a_d1)
    a_part_s = mm(hp, a_s1)
    al_self, al_part = _pair_attn(a_self_s, a_self_d, a_part_s)
    out1 = _elu(mm(al_self, e4) * h + mm(al_part, e4) * hp + b1_ref[...])

    h2 = mm(out1, w2_ref[...])
    h2p = _pair_swap(h2)
    a2_self_s = mm(h2, as2_ref[...])
    a2_self_d = mm(h2, ad2_ref[...])
    a2_part_s = mm(h2p, as2_ref[...])
    a2_self, a2_part = _pair_attn(a2_self_s, a2_self_d, a2_part_s)
    out_ref[...] = a2_self * h2 + a2_part * h2p + b2_ref[...]


def kernel(rna_node_features, rna_batch_idx, rna_dot_bracket_codes,
           W1, a_src1, a_dst1, b1, W2, a_src2, a_dst2, b2):
    B, L = rna_dot_bracket_codes.shape
    D = rna_node_features.shape[1]
    heads, out1 = a_src1.shape
    nslab = B // _G
    x_s = rna_node_features.reshape(nslab, _G * L, D)

    eyeh = jnp.eye(heads, dtype=jnp.float32)
    A_s1 = (eyeh[:, None, :] * a_src1[:, :, None]).reshape(heads * out1, heads)
    A_d1 = (eyeh[:, None, :] * a_dst1[:, :, None]).reshape(heads * out1, heads)
    E4 = jnp.repeat(eyeh, out1, axis=1)

    Gb = 25
    grid = (nslab // Gb,)
    codes_g = rna_dot_bracket_codes.reshape(nslab // Gb, Gb, _G * L)

    out = pl.pallas_call(
        partial(_fused_kernel, L),
        grid=grid,
        in_specs=[
            pl.BlockSpec((Gb, _G * L, D), lambda i: (i, 0, 0)),
            pl.BlockSpec((1, Gb, _G * L), lambda i: (i, 0, 0)),
            pl.BlockSpec((D, D), lambda i: (0, 0)),
            pl.BlockSpec((D, heads), lambda i: (0, 0)),
            pl.BlockSpec((D, heads), lambda i: (0, 0)),
            pl.BlockSpec((1, D), lambda i: (0, 0)),
            pl.BlockSpec((D, D), lambda i: (0, 0)),
            pl.BlockSpec((D, 1), lambda i: (0, 0)),
            pl.BlockSpec((D, 1), lambda i: (0, 0)),
            pl.BlockSpec((1, D), lambda i: (0, 0)),
            pl.BlockSpec((heads, D), lambda i: (0, 0)),
        ],
        out_specs=pl.BlockSpec((2 * Gb * _G, D), lambda i: (i, 0)),
        out_shape=jax.ShapeDtypeStruct((2 * B, D), jnp.float32),
    )(x_s, codes_g, W1, A_s1, A_d1, b1.reshape(1, D),
      W2, a_src2.reshape(D, 1), a_dst2.reshape(D, 1), b2.reshape(1, D), E4)

    motif_batch_idx = jnp.repeat(jnp.arange(B), 2)
    return (out, motif_batch_idx)


# speedup vs baseline: 3.8809x; 1.1852x over previous
import jax
import jax.numpy as jnp
from jax import lax
from jax.experimental import pallas as pl
from functools import partial

_G = 8


def _leaky(x):
    return jnp.where(x >= 0, x, 0.2 * x)


def _elu(x):
    return jnp.where(x > 0, x, jnp.exp(jnp.minimum(x, 0.0)) - 1.0)


def _pair_swap(v):
    even = (lax.broadcasted_iota(jnp.int32, v.shape, 0) & 1) == 0
    return jnp.where(even, jnp.roll(v, -1, axis=0), jnp.roll(v, 1, axis=0))


def _pair_attn(a_self_src, a_self_dst, a_part_src):
    e_self = _leaky(a_self_src + a_self_dst)
    e_part = _leaky(a_part_src + a_self_dst)
    m = jnp.maximum(e_self, e_part)
    ex_s = jnp.exp(e_self - m)
    ex_p = jnp.exp(e_part - m)
    s = ex_s + ex_p + 1e-16
    return ex_s / s, ex_p / s


def _fused_kernel(L, x_ref, codes_ref, seg_ref, segt_ref, segi_ref,
                  w1_ref, as1_ref, ad1_ref, b1_ref,
                  w2_ref, as2_ref, ad2_ref, b2_ref, e4_ref, out_ref):
    x = x_ref[...]
    Gb, GL, D = x.shape
    codes = codes_ref[0]
    mm = partial(jnp.dot, preferred_element_type=jnp.float32)

    m = (codes != 0).astype(jnp.float32)
    cnt = mm(m, segt_ref[...])
    cntb = mm(cnt, seg_ref[...])
    wS = m / jnp.maximum(cntb, 1.0)
    wL = (1.0 - m) / jnp.maximum(L - cntb, 1.0)
    segi = segi_ref[...]
    par = (lax.broadcasted_iota(jnp.int32, (2 * _G, GL), 0) & 1) == 0
    pieces = []
    for g in range(Gb):
        wSg = jnp.broadcast_to(wS[g][None, :], (2 * _G, GL))
        wLg = jnp.broadcast_to(wL[g][None, :], (2 * _G, GL))
        Mw = segi * jnp.where(par, wSg, wLg)
        pieces.append(mm(Mw, x[g]))
    init = jnp.concatenate(pieces, axis=0)

    w1 = w1_ref[...]
    a_s1 = as1_ref[...]
    a_d1 = ad1_ref[...]
    e4 = e4_ref[...]

    h = mm(init, w1)
    hp = _pair_swap(h)
    a_self_s = mm(h, a_s1)
    a_self_d = mm(h, a_d1)
    a_part_s = mm(hp, a_s1)
    al_self, al_part = _pair_attn(a_self_s, a_self_d, a_part_s)
    out1 = _elu(mm(al_self, e4) * h + mm(al_part, e4) * hp + b1_ref[...])

    h2 = mm(out1, w2_ref[...])
    h2p = _pair_swap(h2)
    a2_self_s = mm(h2, as2_ref[...])
    a2_self_d = mm(h2, ad2_ref[...])
    a2_part_s = mm(h2p, as2_ref[...])
    a2_self, a2_part = _pair_attn(a2_self_s, a2_self_d, a2_part_s)
    out_ref[...] = a2_self * h2 + a2_part * h2p + b2_ref[...]


def kernel(rna_node_features, rna_batch_idx, rna_dot_bracket_codes,
           W1, a_src1, a_dst1, b1, W2, a_src2, a_dst2, b2):
    B, L = rna_dot_bracket_codes.shape
    D = rna_node_features.shape[1]
    heads, out1 = a_src1.shape
    nslab = B // _G
    x_s = rna_node_features.reshape(nslab, _G * L, D)

    eyeh = jnp.eye(heads, dtype=jnp.float32)
    A_s1 = (eyeh[:, None, :] * a_src1[:, :, None]).reshape(heads * out1, heads)
    A_d1 = (eyeh[:, None, :] * a_dst1[:, :, None]).reshape(heads * out1, heads)
    E4 = jnp.repeat(eyeh, out1, axis=1)

    SEG = jnp.repeat(jnp.eye(_G, dtype=jnp.float32), L, axis=1)
    SEGI = jnp.repeat(SEG, 2, axis=0)

    Gb = 25
    grid = (nslab // Gb,)
    codes_g = rna_dot_bracket_codes.reshape(nslab // Gb, Gb, _G * L)

    out = pl.pallas_call(
        partial(_fused_kernel, L),
        grid=grid,
        in_specs=[
            pl.BlockSpec((Gb, _G * L, D), lambda i: (i, 0, 0)),
            pl.BlockSpec((1, Gb, _G * L), lambda i: (i, 0, 0)),
            pl.BlockSpec((_G, _G * L), lambda i: (0, 0)),
            pl.BlockSpec((_G * L, _G), lambda i: (0, 0)),
            pl.BlockSpec((2 * _G, _G * L), lambda i: (0, 0)),
            pl.BlockSpec((D, D), lambda i: (0, 0)),
            pl.BlockSpec((D, heads), lambda i: (0, 0)),
            pl.BlockSpec((D, heads), lambda i: (0, 0)),
            pl.BlockSpec((1, D), lambda i: (0, 0)),
            pl.BlockSpec((D, D), lambda i: (0, 0)),
            pl.BlockSpec((D, 1), lambda i: (0, 0)),
            pl.BlockSpec((D, 1), lambda i: (0, 0)),
            pl.BlockSpec((1, D), lambda i: (0, 0)),
            pl.BlockSpec((heads, D), lambda i: (0, 0)),
        ],
        out_specs=pl.BlockSpec((2 * Gb * _G, D), lambda i: (i, 0)),
        out_shape=jax.ShapeDtypeStruct((2 * B, D), jnp.float32),
    )(x_s, codes_g, SEG, SEG.T, SEGI, W1, A_s1, A_d1, b1.reshape(1, D),
      W2, a_src2.reshape(D, 1), a_dst2.reshape(D, 1), b2.reshape(1, D), E4)

    motif_batch_idx = jnp.repeat(jnp.arange(B), 2)
    return (out, motif_batch_idx)
